# Initial kernel scaffold; baseline (speedup 1.0000x reference)
import jax
import jax.numpy as jnp
from jax.experimental import pallas as pl


def kernel(x, Wr, temp, Wg, Wu, Wd):
    raise NotImplementedError("write your pallas kernel here")


# speedup vs baseline: 3.5137x; 3.5137x over previous
import jax
import jax.numpy as jnp
from jax.experimental import pallas as pl
from jax.experimental.pallas import tpu as pltpu

_B, _S, _D = 1, 2048, 768
_E, _K = 8, 2
_H = _D * 2
_CF = 1.25
_AUX = 0.01
_T = _B * _S
_C = max(1, int(_T * _K / _E * _CF))


def _router_kernel(x_ref, wr_ref, t_ref, pos_ref, coef_ref, aux_ref, ent_ref,
                   ovf_ref):
    xf = x_ref[...]
    wr = wr_ref[...]
    t = jnp.clip(t_ref[0, 0], 0.1, 5.0)
    logits = jax.lax.dot_general(
        xf, wr, (((1,), (1,)), ((), ())),
        preferred_element_type=jnp.float32) / t
    m = jnp.max(logits, axis=1, keepdims=True)
    ex = jnp.exp(logits - m)
    probs = ex / jnp.sum(ex, axis=1, keepdims=True)

    eio = jax.lax.broadcasted_iota(jnp.int32, (_T, _E), 1)
    m0 = jnp.max(probs, axis=1, keepdims=True)
    i0 = jnp.min(jnp.where(probs == m0, eio, _E), axis=1, keepdims=True)
    p2 = jnp.where(eio == i0, -jnp.inf, probs)
    m1 = jnp.max(p2, axis=1, keepdims=True)
    i1 = jnp.min(jnp.where(p2 == m1, eio, _E), axis=1, keepdims=True)
    mask = ((eio == i0) | (eio == i1)).astype(jnp.float32)

    tio = jax.lax.broadcasted_iota(jnp.int32, (_T, _E), 0)
    cum = mask
    s = 1
    while s < _T:
        sh = pltpu.roll(cum, s, axis=0)
        cum = cum + jnp.where(tio >= s, sh, 0.0)
        s *= 2

    keep = mask * (cum <= _C).astype(jnp.float32)
    pos_ref[...] = jnp.where(keep > 0, cum - 1.0, -1.0)
    coef_ref[...] = probs * keep

    importance = jnp.sum(probs, axis=0, keepdims=True) / _T
    load = jnp.sum(mask, axis=0, keepdims=True) / (_T + 1e-06)
    aux = jnp.sum(importance * load, axis=1, keepdims=True) * _E * _AUX
    plogp = probs * jnp.log(jnp.clip(probs, 1e-08))
    ent = -jnp.sum(plogp, axis=1, keepdims=True)
    ent = jnp.sum(ent, axis=0, keepdims=True) / _T * 0.01
    nmask = jnp.sum(mask, axis=0, keepdims=True)
    nkeep = jnp.sum(keep, axis=0, keepdims=True)
    tot_mask = jnp.sum(nmask, axis=1, keepdims=True)
    tot_drop = jnp.sum(nmask - nkeep, axis=1, keepdims=True)
    ovf = tot_drop / jnp.maximum(tot_mask, 1.0)
    aux_ref[...] = aux
    ent_ref[...] = ent
    ovf_ref[...] = ovf


def _moe_kernel(pos_ref, coef_ref, x_ref, wg_ref, wu_ref, wd_ref, o_ref):
    e = pl.program_id(0)
    pe = pos_ref[0]
    cf = coef_ref[0]
    ci = jax.lax.broadcasted_iota(jnp.int32, (_C, _T), 0)
    pei = jnp.broadcast_to(pe, (_C, _T)).astype(jnp.int32)
    P = (pei == ci).astype(jnp.float32)

    x = x_ref[...]
    buf = jnp.dot(P, x, preferred_element_type=jnp.float32)
    wg = wg_ref[0]
    wu = wu_ref[0]
    wd = wd_ref[0]
    g = jax.lax.dot_general(buf, wg, (((1,), (1,)), ((), ())),
                            preferred_element_type=jnp.float32)
    u = jax.lax.dot_general(buf, wu, (((1,), (1,)), ((), ())),
                            preferred_element_type=jnp.float32)
    hid = (u * jax.nn.sigmoid(u)) * g
    ob = jax.lax.dot_general(hid, wd, (((1,), (1,)), ((), ())),
                             preferred_element_type=jnp.float32)
    A = P * cf
    contrib = jax.lax.dot_general(A, ob, (((0,), (0,)), ((), ())),
                                  preferred_element_type=jnp.float32)

    @pl.when(e == 0)
    def _():
        o_ref[...] = contrib

    @pl.when(e != 0)
    def _():
        o_ref[...] += contrib


def kernel(x, Wr, temp, Wg, Wu, Wd):
    xf = x.reshape(_T, _D)
    t2 = temp.reshape(1, 1)
    s11 = jax.ShapeDtypeStruct((1, 1), jnp.float32)
    pos_te, coef_te, aux, ent, ovf = pl.pallas_call(
        _router_kernel,
        out_shape=(
            jax.ShapeDtypeStruct((_T, _E), jnp.float32),
            jax.ShapeDtypeStruct((_T, _E), jnp.float32),
            s11, s11, s11,
        ),
    )(xf, Wr, t2)

    pos3 = pos_te.T.reshape(_E, 1, _T)
    coef3 = coef_te.T.reshape(_E, 1, _T)

    out = pl.pallas_call(
        _moe_kernel,
        grid=(_E,),
        in_specs=[
            pl.BlockSpec((1, 1, _T), lambda e: (e, 0, 0)),
            pl.BlockSpec((1, 1, _T), lambda e: (e, 0, 0)),
            pl.BlockSpec((_T, _D), lambda e: (0, 0)),
            pl.BlockSpec((1, _H, _D), lambda e: (e, 0, 0)),
            pl.BlockSpec((1, _H, _D), lambda e: (e, 0, 0)),
            pl.BlockSpec((1, _D, _H), lambda e: (e, 0, 0)),
        ],
        out_specs=pl.BlockSpec((_T, _D), lambda e: (0, 0)),
        out_shape=jax.ShapeDtypeStruct((_T, _D), jnp.float32),
        compiler_params=pltpu.CompilerParams(
            dimension_semantics=("arbitrary",)),
    )(pos3, coef3, xf, Wg, Wu, Wd)

    return (out.reshape(_B, _S, _D), aux[0, 0], ent[0, 0], ovf[0, 0])
